# baseline (speedup 1.0000x reference)
import functools

import jax
import jax.numpy as jnp
from jax.experimental import pallas as pl
from jax.experimental.pallas import tpu as pltpu

_BM = 64


_NEG = -1e30


@functools.cache
def _gumbel_interleaved(batch: int, n: int):
    g = jax.random.gumbel(jax.random.key(42), (batch, n), dtype=jnp.float32)
    return jnp.stack([g, jnp.full_like(g, _NEG)], axis=-1).reshape(batch, 2 * n)


def _body(x_ref, g_ref, w1_ref, b1_ref, w2_ref, b2_ref, o_ref):
    x = x_ref[...]
    y = x * x
    ln = x.shape[1]
    u = y + pltpu.roll(y, ln - 1, 1)
    t = u + g_ref[...]
    mx = jnp.max(t, axis=1, keepdims=True)
    e = jnp.exp(2.0 * (t - mx))
    na = jnp.dot(e, w1_ref[...], preferred_element_type=jnp.float32)
    h = jnp.maximum(na[:, :-1] / na[:, -1:] + b1_ref[...], 0.0)
    o_ref[...] = (
        jnp.dot(h, w2_ref[...], preferred_element_type=jnp.float32) + b2_ref[...]
    )


def kernel(quantum_state, W1, b1, W2, b2):
    batch, n, _ = quantum_state.shape
    odim = W2.shape[1]
    x2 = quantum_state.reshape(batch, 2 * n)
    gd = _gumbel_interleaved(batch, n)
    w1a = jnp.concatenate(
        [jnp.repeat(W1, 2, axis=0), jnp.ones((2 * n, 1), jnp.float32)], axis=1
    )
    bm = min(_BM, batch)

    return pl.pallas_call(
        _body,
        grid=(batch // bm,),
        in_specs=[
            pl.BlockSpec((bm, 2 * n), lambda i: (i, 0)),
            pl.BlockSpec((bm, 2 * n), lambda i: (i, 0)),
            pl.BlockSpec((2 * n, W1.shape[1] + 1), lambda i: (0, 0)),
            pl.BlockSpec((1, W1.shape[1]), lambda i: (0, 0)),
            pl.BlockSpec(W2.shape, lambda i: (0, 0)),
            pl.BlockSpec((1, odim), lambda i: (0, 0)),
        ],
        out_specs=pl.BlockSpec((bm, odim), lambda i: (i, 0)),
        out_shape=jax.ShapeDtypeStruct((batch, odim), jnp.float32),
        compiler_params=pltpu.CompilerParams(
            dimension_semantics=("arbitrary",),
        ),
    )(x2, gd, w1a, b1.reshape(1, -1), W2, b2.reshape(1, -1))


# speedup vs baseline: 3.2663x; 3.2663x over previous
import functools

import jax
import jax.numpy as jnp
from jax.experimental import pallas as pl
from jax.experimental.pallas import tpu as pltpu

_BM = 64


@functools.cache
def _gumbel_const(batch: int, n: int):
    return jax.random.gumbel(jax.random.key(42), (batch, n), dtype=jnp.float32)


def _body(x_ref, g_ref, w1_ref, b1_ref, w2_ref, b2_ref, o_ref):
    x = x_ref[...]
    y = x * x
    n = y.shape[1] // 2
    t = y[:, :n] + y[:, n:] + g_ref[...]
    mx = jnp.max(t, axis=1, keepdims=True)
    e = jnp.exp(2.0 * (t - mx))
    s = jnp.sum(e, axis=1, keepdims=True)
    num = jnp.dot(e, w1_ref[...], preferred_element_type=jnp.float32)
    h = jnp.maximum(num / s + b1_ref[...], 0.0)
    o_ref[...] = (
        jnp.dot(h, w2_ref[...], preferred_element_type=jnp.float32) + b2_ref[...]
    )


def kernel(quantum_state, W1, b1, W2, b2):
    batch, n, _ = quantum_state.shape
    odim = W2.shape[1]
    xp = jnp.transpose(quantum_state, (0, 2, 1)).reshape(batch, 2 * n)
    g = _gumbel_const(batch, n)
    bm = min(_BM, batch)

    return pl.pallas_call(
        _body,
        grid=(batch // bm,),
        in_specs=[
            pl.BlockSpec((bm, 2 * n), lambda i: (i, 0)),
            pl.BlockSpec((bm, n), lambda i: (i, 0)),
            pl.BlockSpec((n, W1.shape[1]), lambda i: (0, 0)),
            pl.BlockSpec((1, W1.shape[1]), lambda i: (0, 0)),
            pl.BlockSpec(W2.shape, lambda i: (0, 0)),
            pl.BlockSpec((1, odim), lambda i: (0, 0)),
        ],
        out_specs=pl.BlockSpec((bm, odim), lambda i: (i, 0)),
        out_shape=jax.ShapeDtypeStruct((batch, odim), jnp.float32),
        compiler_params=pltpu.CompilerParams(
            dimension_semantics=("arbitrary",),
        ),
    )(xp, g, W1, b1.reshape(1, -1), W2, b2.reshape(1, -1))


# speedup vs baseline: 3.3508x; 1.0259x over previous
import functools

import jax
import jax.numpy as jnp
from jax.experimental import pallas as pl
from jax.experimental.pallas import tpu as pltpu

_BM = 128


@functools.cache
def _gumbel_const(batch: int, n: int):
    return jax.random.gumbel(jax.random.key(42), (batch, n), dtype=jnp.float32)


def _body(x_ref, g_ref, w1_ref, b1_ref, w2_ref, b2_ref, o_ref):
    x = x_ref[...]
    y = x * x
    n = y.shape[1] // 2
    t = y[:, :n] + y[:, n:] + g_ref[...]
    mx = jnp.max(t, axis=1, keepdims=True)
    e = jnp.exp(2.0 * (t - mx))
    s = jnp.sum(e, axis=1, keepdims=True)
    num = jnp.dot(e, w1_ref[...], preferred_element_type=jnp.float32)
    h = jnp.maximum(num / s + b1_ref[...], 0.0)
    o_ref[...] = (
        jnp.dot(h, w2_ref[...], preferred_element_type=jnp.float32) + b2_ref[...]
    )


def kernel(quantum_state, W1, b1, W2, b2):
    batch, n, _ = quantum_state.shape
    odim = W2.shape[1]
    xp = jnp.transpose(quantum_state, (0, 2, 1)).reshape(batch, 2 * n)
    g = _gumbel_const(batch, n)
    bm = min(_BM, batch)

    return pl.pallas_call(
        _body,
        grid=(batch // bm,),
        in_specs=[
            pl.BlockSpec((bm, 2 * n), lambda i: (i, 0)),
            pl.BlockSpec((bm, n), lambda i: (i, 0)),
            pl.BlockSpec((n, W1.shape[1]), lambda i: (0, 0)),
            pl.BlockSpec((1, W1.shape[1]), lambda i: (0, 0)),
            pl.BlockSpec(W2.shape, lambda i: (0, 0)),
            pl.BlockSpec((1, odim), lambda i: (0, 0)),
        ],
        out_specs=pl.BlockSpec((bm, odim), lambda i: (i, 0)),
        out_shape=jax.ShapeDtypeStruct((batch, odim), jnp.float32),
        compiler_params=pltpu.CompilerParams(
            dimension_semantics=("arbitrary",),
        ),
    )(xp, g, W1, b1.reshape(1, -1), W2, b2.reshape(1, -1))
